# baseline (speedup 1.0000x reference)
import functools

import jax
import jax.numpy as jnp
from jax import lax
from jax.experimental import pallas as pl
from jax.experimental.pallas import tpu as pltpu
from jax.experimental.pallas import tpu_sc as plsc

N = 10000
NP = 10240
E = 320000
ROWS_PER_TILE = NP // 16
EPS = 1e-5

_MESH = plsc.VectorSubcoreMesh(core_axis_name="c", subcore_axis_name="s")
_SC_PARAMS = pltpu.CompilerParams(use_tc_tiling_on_sc=False)


EPT = E // 32


def _make_agg_kernel(d, kk):
    cpt = EPT // kk

    @functools.partial(
        pl.kernel,
        mesh=_MESH,
        out_type=jax.ShapeDtypeStruct((2, NP, d), jnp.float32),
        scratch_types=[
            pltpu.VMEM((EPT,), jnp.int32),
            pltpu.VMEM((EPT,), jnp.int32),
            pltpu.VMEM((kk, d), jnp.float32),
            pltpu.VMEM((kk, d), jnp.float32),
            pltpu.VMEM_SHARED((NP, d), jnp.float32),
            pltpu.SemaphoreType.DMA,
            pltpu.SemaphoreType.DMA,
            pltpu.SemaphoreType.DMA,
            pltpu.SemaphoreType.DMA,
        ],
        compiler_params=_SC_PARAMS,
    )
    def agg_kernel(h_hbm, src_hbm, dst_hbm, zeros_hbm, out_hbm, src_v, dst_v,
                   buf0, buf1, acc, sem0, sem1, ssem0, ssem1):
        cid = lax.axis_index("c")
        sid = lax.axis_index("s")
        wid = cid * 16 + sid
        bufs = (buf0, buf1)
        sems = (sem0, sem1)
        ssems = (ssem0, ssem1)

        pltpu.sync_copy(src_hbm.at[pl.ds(wid * EPT, EPT)], src_v)
        pltpu.sync_copy(dst_hbm.at[pl.ds(wid * EPT, EPT)], dst_v)
        base = sid * ROWS_PER_TILE
        pltpu.sync_copy(zeros_hbm, acc.at[pl.ds(base, ROWS_PER_TILE)])
        plsc.subcore_barrier()

        def src_at(j):
            return src_v.at[pl.ds(j * kk, kk)]

        def dst_at(j):
            return dst_v.at[pl.ds(j * kk, kk)]

        pltpu.async_copy(h_hbm.at[src_at(0)], buf0, sem0)

        @pl.loop(0, cpt, step=2)
        def _(j0):
            for u in range(2):
                j = j0 + u

                @pl.when(j >= 1)
                def _():
                    pltpu.make_async_copy(bufs[1 - u], acc.at[dst_at(j - 1)],
                                          ssems[1 - u]).wait()

                @pl.when(j + 1 < cpt)
                def _():
                    pltpu.async_copy(h_hbm.at[src_at(j + 1)], bufs[1 - u],
                                     sems[1 - u])

                pltpu.make_async_copy(h_hbm.at[src_at(j)], bufs[u],
                                      sems[u]).wait()
                pltpu.async_copy(bufs[u], acc.at[dst_at(j)], ssems[u],
                                 add=True)

        pltpu.make_async_copy(
            bufs[(cpt - 1) % 2], acc.at[dst_at(cpt - 1)],
            ssems[(cpt - 1) % 2]).wait()
        plsc.subcore_barrier()

        pltpu.sync_copy(acc.at[pl.ds(base, ROWS_PER_TILE)],
                        out_hbm.at[cid, pl.ds(base, ROWS_PER_TILE)])

    return agg_kernel


_agg64 = _make_agg_kernel(64, 200)
_agg16 = _make_agg_kernel(16, 200)


def _tc_matmul_body(x_ref, w_ref, o_ref):
    o_ref[...] = jnp.dot(x_ref[...], w_ref[...],
                         preferred_element_type=jnp.float32)


def _tc_prep_body(h_ref, degp_ref, dinv_ref, lo_ref, hi_ref):
    deg = degp_ref[0, :N, 0:1] + degp_ref[1, :N, 0:1] + 1.0
    dinv = lax.rsqrt(deg)
    dinv_ref[...] = dinv
    hp = h_ref[...] * dinv
    lo_ref[...] = hp[:, :64]
    hi_ref[...] = hp[:, 64:]


def _bn_relu_mm_scale(y, g, be, w, dinv):
    mean = jnp.mean(y, axis=0, keepdims=True)
    var = jnp.mean((y - mean) ** 2, axis=0, keepdims=True)
    t = g * (y - mean) * lax.rsqrt(var + EPS) + be
    t = jnp.maximum(t, 0.0)
    return jnp.dot(t, w, preferred_element_type=jnp.float32) * dinv


def _tc_layer1_body(plo_ref, phi_ref, lo_ref, hi_ref, dinv_ref, b_ref, g_ref,
                    be_ref, w_ref, o_ref):
    dinv = dinv_ref[...]
    agg = jnp.concatenate(
        [plo_ref[0, :N] + plo_ref[1, :N] + lo_ref[...],
         phi_ref[0, :N] + phi_ref[1, :N] + hi_ref[...]], axis=1)
    y = agg * dinv + b_ref[...]
    o_ref[...] = _bn_relu_mm_scale(y, g_ref[...], be_ref[...], w_ref[...], dinv)


def _tc_layer2_body(p_ref, hp_ref, dinv_ref, b_ref, g_ref, be_ref, w_ref, o_ref):
    dinv = dinv_ref[...]
    y = (p_ref[0, :N] + p_ref[1, :N] + hp_ref[...]) * dinv + b_ref[...]
    o_ref[...] = _bn_relu_mm_scale(y, g_ref[...], be_ref[...], w_ref[...], dinv)


def _tc_final_body(p_ref, hp_ref, dinv_ref, b_ref, o_ref):
    o_ref[...] = ((p_ref[0, :N] + p_ref[1, :N] + hp_ref[...])
                  * dinv_ref[...] + b_ref[...])


def _tc_call(body, out_shape, *args):
    return pl.pallas_call(body, out_shape=out_shape)(*args)


def kernel(x, edge_index, W1, b1, g1, be1, W2, b2, g2, be2, W3, b3):
    f32 = jnp.float32
    src = edge_index[0]
    dst = edge_index[1]
    z64 = jnp.zeros((ROWS_PER_TILE, 64), f32)
    z16 = jnp.zeros((ROWS_PER_TILE, 16), f32)

    ones_tab = jnp.ones((16, 16), f32)
    zsrc = jnp.zeros((E,), edge_index.dtype)
    degp = _agg16(ones_tab, zsrc, dst, z16)
    h1r = _tc_call(_tc_matmul_body, jax.ShapeDtypeStruct((N, 128), f32), x, W1)

    dinv, h1lo, h1hi = _tc_call(
        _tc_prep_body,
        (jax.ShapeDtypeStruct((N, 1), f32),
         jax.ShapeDtypeStruct((N, 64), f32),
         jax.ShapeDtypeStruct((N, 64), f32)),
        h1r, degp)

    p1lo = _agg64(h1lo, src, dst, z64)
    p1hi = _agg64(h1hi, src, dst, z64)
    h2p = _tc_call(_tc_layer1_body, jax.ShapeDtypeStruct((N, 64), f32),
                   p1lo, p1hi, h1lo, h1hi, dinv, b1.reshape(1, 128),
                   g1.reshape(1, 128), be1.reshape(1, 128), W2)

    p2 = _agg64(h2p, src, dst, z64)
    W3p = jnp.pad(W3, ((0, 0), (0, 14)))
    h3p = _tc_call(_tc_layer2_body, jax.ShapeDtypeStruct((N, 16), f32),
                   p2, h2p, dinv, b2.reshape(1, 64), g2.reshape(1, 64),
                   be2.reshape(1, 64), W3p)

    p3 = _agg16(h3p, src, dst, z16)
    b3p = jnp.pad(b3, (0, 14)).reshape(1, 16)
    out16 = _tc_call(_tc_final_body, jax.ShapeDtypeStruct((N, 16), f32),
                     p3, h3p, dinv, b3p)
    return out16[:, :2]


# speedup vs baseline: 4.7683x; 4.7683x over previous
import functools

import jax
import jax.numpy as jnp
from jax import lax
from jax.experimental import pallas as pl
from jax.experimental.pallas import tpu as pltpu
from jax.experimental.pallas import tpu_sc as plsc

N = 10000
NP = 10240
E = 320000
ROWS_PER_TILE = NP // 16
EPS = 1e-5

_MESH = plsc.VectorSubcoreMesh(core_axis_name="c", subcore_axis_name="s")
_SC_PARAMS = pltpu.CompilerParams(use_tc_tiling_on_sc=False)


EPT = E // 32


def _make_agg_kernel(d, kk):
    cpt = EPT // kk

    @functools.partial(
        pl.kernel,
        mesh=_MESH,
        out_type=jax.ShapeDtypeStruct((2, NP, d), jnp.float32),
        scratch_types=[
            pltpu.VMEM((EPT,), jnp.int32),
            pltpu.VMEM((EPT,), jnp.int32),
            pltpu.VMEM((kk, d), jnp.float32),
            pltpu.VMEM((kk, d), jnp.float32),
            pltpu.VMEM_SHARED((NP, d), jnp.float32),
            pltpu.SemaphoreType.DMA,
            pltpu.SemaphoreType.DMA,
            pltpu.SemaphoreType.DMA,
            pltpu.SemaphoreType.DMA,
        ],
        compiler_params=_SC_PARAMS,
    )
    def agg_kernel(h_hbm, src_hbm, dst_hbm, zeros_hbm, out_hbm, src_v, dst_v,
                   buf0, buf1, acc, sem0, sem1, ssem0, ssem1):
        cid = lax.axis_index("c")
        sid = lax.axis_index("s")
        wid = cid * 16 + sid
        bufs = (buf0, buf1)
        sems = (sem0, sem1)
        ssems = (ssem0, ssem1)

        pltpu.sync_copy(src_hbm.at[pl.ds(wid * EPT, EPT)], src_v)
        pltpu.sync_copy(dst_hbm.at[pl.ds(wid * EPT, EPT)], dst_v)
        base = sid * ROWS_PER_TILE
        pltpu.sync_copy(zeros_hbm, acc.at[pl.ds(base, ROWS_PER_TILE)])
        plsc.subcore_barrier()

        def src_at(j):
            return src_v.at[pl.ds(j * kk, kk)]

        def dst_at(j):
            return dst_v.at[pl.ds(j * kk, kk)]

        pltpu.async_copy(h_hbm.at[src_at(0)], buf0, sem0)

        @pl.loop(0, cpt, step=2)
        def _(j0):
            for u in range(2):
                j = j0 + u

                @pl.when(j >= 1)
                def _():
                    pltpu.make_async_copy(bufs[1 - u], acc.at[dst_at(j - 1)],
                                          ssems[1 - u]).wait()

                @pl.when(j + 1 < cpt)
                def _():
                    pltpu.async_copy(h_hbm.at[src_at(j + 1)], bufs[1 - u],
                                     sems[1 - u])

                pltpu.make_async_copy(h_hbm.at[src_at(j)], bufs[u],
                                      sems[u]).wait()
                pltpu.async_copy(bufs[u], acc.at[dst_at(j)], ssems[u],
                                 add=True)

        pltpu.make_async_copy(
            bufs[(cpt - 1) % 2], acc.at[dst_at(cpt - 1)],
            ssems[(cpt - 1) % 2]).wait()
        plsc.subcore_barrier()

        pltpu.sync_copy(acc.at[pl.ds(base, ROWS_PER_TILE)],
                        out_hbm.at[cid, pl.ds(base, ROWS_PER_TILE)])

    return agg_kernel


_agg64 = _make_agg_kernel(64, 200)
_agg16 = _make_agg_kernel(16, 200)


def _tc_matmul_body(x_ref, w_ref, o_ref):
    o_ref[...] = jnp.dot(x_ref[...], w_ref[...],
                         preferred_element_type=jnp.float32)


def _tc_prep_body(h_ref, degp_ref, dinv_ref, lo_ref, hi_ref):
    deg = degp_ref[0, :N, 0:1] + degp_ref[1, :N, 0:1] + 1.0
    dinv = lax.rsqrt(deg)
    dinv_ref[...] = dinv
    hp = h_ref[...] * dinv
    lo_ref[...] = hp[:, :64]
    hi_ref[...] = hp[:, 64:]


def _bn_relu_mm_scale(y, g, be, w, dinv):
    mean = jnp.mean(y, axis=0, keepdims=True)
    var = jnp.mean((y - mean) ** 2, axis=0, keepdims=True)
    t = g * (y - mean) * lax.rsqrt(var + EPS) + be
    t = jnp.maximum(t, 0.0)
    return jnp.dot(t, w, preferred_element_type=jnp.float32) * dinv


def _tc_layer1_body(plo_ref, phi_ref, lo_ref, hi_ref, dinv_ref, b_ref, g_ref,
                    be_ref, w_ref, o_ref):
    dinv = dinv_ref[...]
    agg = jnp.concatenate(
        [plo_ref[0, :N] + plo_ref[1, :N] + lo_ref[...],
         phi_ref[0, :N] + phi_ref[1, :N] + hi_ref[...]], axis=1)
    y = agg * dinv + b_ref[...]
    o_ref[...] = _bn_relu_mm_scale(y, g_ref[...], be_ref[...], w_ref[...], dinv)


def _tc_layer2_body(p_ref, hp_ref, dinv_ref, b_ref, g_ref, be_ref, w_ref, o_ref):
    dinv = dinv_ref[...]
    y = (p_ref[0, :N] + p_ref[1, :N] + hp_ref[...]) * dinv + b_ref[...]
    o_ref[...] = _bn_relu_mm_scale(y, g_ref[...], be_ref[...], w_ref[...], dinv)


def _tc_final_body(p_ref, hp_ref, dinv_ref, b_ref, o_ref):
    o_ref[...] = ((p_ref[0, :N] + p_ref[1, :N] + hp_ref[...])
                  * dinv_ref[...] + b_ref[...])


def _tc_call(body, out_shape, *args):
    return pl.pallas_call(body, out_shape=out_shape)(*args)


def kernel(x, edge_index, W1, b1, g1, be1, W2, b2, g2, be2, W3, b3):
    f32 = jnp.float32
    src = edge_index[0]
    dst = edge_index[1]
    z64 = jnp.zeros((ROWS_PER_TILE, 64), f32)
    z16 = jnp.zeros((ROWS_PER_TILE, 16), f32)

    ones_tab = jnp.ones((N, 16), f32)
    degp = _agg16(ones_tab, src, dst, z16)
    h1r = _tc_call(_tc_matmul_body, jax.ShapeDtypeStruct((N, 128), f32), x, W1)

    dinv, h1lo, h1hi = _tc_call(
        _tc_prep_body,
        (jax.ShapeDtypeStruct((N, 1), f32),
         jax.ShapeDtypeStruct((N, 64), f32),
         jax.ShapeDtypeStruct((N, 64), f32)),
        h1r, degp)

    p1lo = _agg64(h1lo, src, dst, z64)
    p1hi = _agg64(h1hi, src, dst, z64)
    h2p = _tc_call(_tc_layer1_body, jax.ShapeDtypeStruct((N, 64), f32),
                   p1lo, p1hi, h1lo, h1hi, dinv, b1.reshape(1, 128),
                   g1.reshape(1, 128), be1.reshape(1, 128), W2)

    p2 = _agg64(h2p, src, dst, z64)
    W3p = jnp.pad(W3, ((0, 0), (0, 14)))
    h3p = _tc_call(_tc_layer2_body, jax.ShapeDtypeStruct((N, 16), f32),
                   p2, h2p, dinv, b2.reshape(1, 64), g2.reshape(1, 64),
                   be2.reshape(1, 64), W3p)

    p3 = _agg16(h3p, src, dst, z16)
    b3p = jnp.pad(b3, (0, 14)).reshape(1, 16)
    out16 = _tc_call(_tc_final_body, jax.ShapeDtypeStruct((N, 16), f32),
                     p3, h3p, dinv, b3p)
    return out16[:, :2]


# speedup vs baseline: 4.9015x; 1.0279x over previous
import functools

import jax
import jax.numpy as jnp
from jax import lax
from jax.experimental import pallas as pl
from jax.experimental.pallas import tpu as pltpu
from jax.experimental.pallas import tpu_sc as plsc

N = 10000
NP = 10240
E = 320000
ROWS_PER_TILE = NP // 16
EPS = 1e-5

_MESH = plsc.VectorSubcoreMesh(core_axis_name="c", subcore_axis_name="s")
_SC_PARAMS = pltpu.CompilerParams(use_tc_tiling_on_sc=False)


EPT = E // 32


def _make_agg_kernel(d, kk):
    cpt = EPT // kk

    @functools.partial(
        pl.kernel,
        mesh=_MESH,
        out_type=jax.ShapeDtypeStruct((2, NP, d), jnp.float32),
        scratch_types=[
            pltpu.VMEM((EPT,), jnp.int32),
            pltpu.VMEM((EPT,), jnp.int32),
            pltpu.VMEM((kk, d), jnp.float32),
            pltpu.VMEM((kk, d), jnp.float32),
            pltpu.VMEM_SHARED((NP, d), jnp.float32),
            pltpu.SemaphoreType.DMA,
            pltpu.SemaphoreType.DMA,
            pltpu.SemaphoreType.DMA,
            pltpu.SemaphoreType.DMA,
        ],
        compiler_params=_SC_PARAMS,
    )
    def agg_kernel(h_hbm, src_hbm, dst_hbm, zeros_hbm, out_hbm, src_v, dst_v,
                   buf0, buf1, acc, sem0, sem1, ssem0, ssem1):
        cid = lax.axis_index("c")
        sid = lax.axis_index("s")
        wid = cid * 16 + sid
        bufs = (buf0, buf1)
        sems = (sem0, sem1)
        ssems = (ssem0, ssem1)

        pltpu.sync_copy(src_hbm.at[pl.ds(wid * EPT, EPT)], src_v)
        pltpu.sync_copy(dst_hbm.at[pl.ds(wid * EPT, EPT)], dst_v)
        base = sid * ROWS_PER_TILE
        pltpu.sync_copy(zeros_hbm, acc.at[pl.ds(base, ROWS_PER_TILE)])
        plsc.subcore_barrier()

        def src_at(j):
            return src_v.at[pl.ds(j * kk, kk)]

        def dst_at(j):
            return dst_v.at[pl.ds(j * kk, kk)]

        pltpu.async_copy(h_hbm.at[src_at(0)], buf0, sem0)

        @pl.loop(0, cpt, step=2)
        def _(j0):
            for u in range(2):
                j = j0 + u

                @pl.when(j >= 1)
                def _():
                    pltpu.make_async_copy(bufs[1 - u], acc.at[dst_at(j - 1)],
                                          ssems[1 - u]).wait()

                @pl.when(j + 1 < cpt)
                def _():
                    pltpu.async_copy(h_hbm.at[src_at(j + 1)], bufs[1 - u],
                                     sems[1 - u])

                pltpu.make_async_copy(h_hbm.at[src_at(j)], bufs[u],
                                      sems[u]).wait()
                pltpu.async_copy(bufs[u], acc.at[dst_at(j)], ssems[u],
                                 add=True)

        pltpu.make_async_copy(
            bufs[(cpt - 1) % 2], acc.at[dst_at(cpt - 1)],
            ssems[(cpt - 1) % 2]).wait()
        plsc.subcore_barrier()

        pltpu.sync_copy(acc.at[pl.ds(base, ROWS_PER_TILE)],
                        out_hbm.at[cid, pl.ds(base, ROWS_PER_TILE)])

    return agg_kernel


_agg64 = _make_agg_kernel(64, 200)
_agg16 = _make_agg_kernel(16, 200)


def _tc_matmul_body(x_ref, w_ref, o_ref):
    o_ref[...] = jnp.dot(x_ref[...], w_ref[...],
                         preferred_element_type=jnp.float32)


def _tc_idx_body(ei_ref, src_ref, dst_ref):
    src_ref[...] = ei_ref[0]
    dst_ref[...] = ei_ref[1]


def _tc_prep_body(h_ref, degp_ref, dinv_ref, lo_ref, hi_ref):
    deg = degp_ref[0, :N, 0:1] + degp_ref[1, :N, 0:1] + 1.0
    dinv = lax.rsqrt(deg)
    dinv_ref[...] = dinv
    hp = h_ref[...] * dinv
    lo_ref[...] = hp[:, :64]
    hi_ref[...] = hp[:, 64:]


def _bn_relu_mm_scale(y, g, be, w, dinv):
    mean = jnp.mean(y, axis=0, keepdims=True)
    var = jnp.mean((y - mean) ** 2, axis=0, keepdims=True)
    t = g * (y - mean) * lax.rsqrt(var + EPS) + be
    t = jnp.maximum(t, 0.0)
    return jnp.dot(t, w, preferred_element_type=jnp.float32) * dinv


def _tc_layer1_body(plo_ref, phi_ref, lo_ref, hi_ref, dinv_ref, b_ref, g_ref,
                    be_ref, w_ref, o_ref):
    dinv = dinv_ref[...]
    agg = jnp.concatenate(
        [plo_ref[0, :N] + plo_ref[1, :N] + lo_ref[...],
         phi_ref[0, :N] + phi_ref[1, :N] + hi_ref[...]], axis=1)
    y = agg * dinv + b_ref[...]
    o_ref[...] = _bn_relu_mm_scale(y, g_ref[...], be_ref[...], w_ref[...], dinv)


def _tc_layer2_body(p_ref, hp_ref, dinv_ref, b_ref, g_ref, be_ref, w_ref, o_ref):
    dinv = dinv_ref[...]
    y = (p_ref[0, :N] + p_ref[1, :N] + hp_ref[...]) * dinv + b_ref[...]
    o_ref[...] = _bn_relu_mm_scale(y, g_ref[...], be_ref[...], w_ref[...], dinv)


def _tc_final_body(p_ref, hp_ref, dinv_ref, b_ref, o_ref):
    o_ref[...] = ((p_ref[0, :N] + p_ref[1, :N] + hp_ref[...])
                  * dinv_ref[...] + b_ref[...])


def _tc_call(body, out_shape, *args):
    return pl.pallas_call(body, out_shape=out_shape)(*args)


def kernel(x, edge_index, W1, b1, g1, be1, W2, b2, g2, be2, W3, b3):
    f32 = jnp.float32
    src, dst = _tc_call(
        _tc_idx_body,
        (jax.ShapeDtypeStruct((E,), edge_index.dtype),
         jax.ShapeDtypeStruct((E,), edge_index.dtype)),
        edge_index)
    z64 = jnp.zeros((ROWS_PER_TILE, 64), f32)
    z16 = jnp.zeros((ROWS_PER_TILE, 16), f32)

    ones_tab = jnp.ones((N, 16), f32)
    degp = _agg16(ones_tab, src, dst, z16)
    h1r = _tc_call(_tc_matmul_body, jax.ShapeDtypeStruct((N, 128), f32), x, W1)

    dinv, h1lo, h1hi = _tc_call(
        _tc_prep_body,
        (jax.ShapeDtypeStruct((N, 1), f32),
         jax.ShapeDtypeStruct((N, 64), f32),
         jax.ShapeDtypeStruct((N, 64), f32)),
        h1r, degp)

    p1lo = _agg64(h1lo, src, dst, z64)
    p1hi = _agg64(h1hi, src, dst, z64)
    h2p = _tc_call(_tc_layer1_body, jax.ShapeDtypeStruct((N, 64), f32),
                   p1lo, p1hi, h1lo, h1hi, dinv, b1.reshape(1, 128),
                   g1.reshape(1, 128), be1.reshape(1, 128), W2)

    p2 = _agg64(h2p, src, dst, z64)
    W3p = jnp.pad(W3, ((0, 0), (0, 14)))
    h3p = _tc_call(_tc_layer2_body, jax.ShapeDtypeStruct((N, 16), f32),
                   p2, h2p, dinv, b2.reshape(1, 64), g2.reshape(1, 64),
                   be2.reshape(1, 64), W3p)

    p3 = _agg16(h3p, src, dst, z16)
    b3p = jnp.pad(b3, (0, 14)).reshape(1, 16)
    out16 = _tc_call(_tc_final_body, jax.ShapeDtypeStruct((N, 16), f32),
                     p3, h3p, dinv, b3p)
    return out16[:, :2]
